# baseline (speedup 1.0000x reference)
import functools

import jax
import jax.numpy as jnp
from jax import lax
from jax.experimental import pallas as pl
from jax.experimental.pallas import tpu as pltpu
from jax.experimental.pallas import tpu_sc as plsc


def kernel(x):
    mesh = plsc.VectorSubcoreMesh(core_axis_name="c", subcore_axis_name="s")

    @functools.partial(
        pl.kernel,
        mesh=mesh,
        out_type=jax.ShapeDtypeStruct((2, 128), jnp.float32),
        scratch_types=[pltpu.VMEM((2, 128), jnp.float32)],
    )
    def _sc_copy(x_hbm, out_hbm, buf):
        first = (lax.axis_index("c") == 0) & (lax.axis_index("s") == 0)

        @pl.when(first)
        def _():
            pltpu.sync_copy(x_hbm.at[1, pl.ds(2, 2)], buf)
            pltpu.sync_copy(buf, out_hbm)

    return _sc_copy(x)


# speedup vs baseline: 1.2713x; 1.2713x over previous
import functools

import jax
import jax.numpy as jnp
from jax import lax
from jax.experimental import pallas as pl
from jax.experimental.pallas import tpu as pltpu
from jax.experimental.pallas import tpu_sc as plsc


def kernel(x):
    mesh = plsc.ScalarSubcoreMesh(axis_name="c", num_cores=1)

    @functools.partial(
        pl.kernel,
        mesh=mesh,
        out_type=jax.ShapeDtypeStruct((2, 128), jnp.float32),
    )
    def _sc_copy(x_hbm, out_hbm):
        pltpu.sync_copy(x_hbm.at[1, pl.ds(2, 2)], out_hbm)

    return _sc_copy(x)
